# baseline (speedup 1.0000x reference)
import functools

import jax
import jax.numpy as jnp
from jax import lax
from jax.experimental import pallas as pl
from jax.experimental.pallas import tpu as pltpu
from jax.experimental.pallas import tpu_sc as plsc

HEAD_SIZE = 128
HALF = 64
NUM_Q_HEADS = 32
NUM_KV_HEADS = 8

_NC, _NS = 2, 16
_NW = _NC * _NS


def _sc_gather(pos2d, cos_cache, sin_cache):
    T = pos2d.shape[0] * 128
    idx_rows = pos2d.shape[0] // _NW
    rows = idx_rows * 128

    def body(pos_hbm, cos_hbm, sin_hbm, cos_out, sin_out,
             idx_v, cbuf, sbuf, sem):
        wid = lax.axis_index("s") * _NC + lax.axis_index("c")
        pltpu.sync_copy(pos_hbm.at[pl.ds(wid * idx_rows, idx_rows)], idx_v)
        copies = []
        for j in range(idx_rows):
            copies.append(pltpu.async_copy(
                cos_hbm.at[idx_v.at[j]], cbuf.at[pl.ds(j * 128, 128)], sem))
            copies.append(pltpu.async_copy(
                sin_hbm.at[idx_v.at[j]], sbuf.at[pl.ds(j * 128, 128)], sem))
        for c in copies:
            c.wait()
        base = wid * rows
        pltpu.sync_copy(cbuf, cos_out.at[pl.ds(base, rows)])
        pltpu.sync_copy(sbuf, sin_out.at[pl.ds(base, rows)])

    mesh = plsc.VectorSubcoreMesh(core_axis_name="c", subcore_axis_name="s",
                                  num_cores=_NC, num_subcores=_NS)
    f = pl.kernel(
        body,
        out_type=[jax.ShapeDtypeStruct((T, HEAD_SIZE), jnp.float32),
                  jax.ShapeDtypeStruct((T, HEAD_SIZE), jnp.float32)],
        mesh=mesh,
        scratch_types=[
            pltpu.VMEM((idx_rows, 128), jnp.int32),
            pltpu.VMEM((rows, HEAD_SIZE), jnp.float32),
            pltpu.VMEM((rows, HEAD_SIZE), jnp.float32),
            pltpu.SemaphoreType.DMA,
        ],
    )
    return f(pos2d, cos_cache, sin_cache)


def _apply_body(cos_ref, sin_ref, q_ref, k_ref, *rest):
    qo_ref, ko_ref = rest[-2], rest[-1]
    c = cos_ref[...][:, :HALF]
    s = sin_ref[...][:, :HALF]
    cc = jnp.concatenate([c, c], axis=1)
    ss = jnp.concatenate([-s, s], axis=1)
    for x_ref, o_ref, heads in ((q_ref, qo_ref, NUM_Q_HEADS),
                                (k_ref, ko_ref, NUM_KV_HEADS)):
        for h in range(heads):
            x = x_ref[:, h * HEAD_SIZE:(h + 1) * HEAD_SIZE]
            xs = jnp.concatenate([x[:, HALF:], x[:, :HALF]], axis=1)
            o_ref[:, h * HEAD_SIZE:(h + 1) * HEAD_SIZE] = x * cc + xs * ss


def _tc_apply_chunk(cos_g, sin_g, q, k, prev, block_t, chunk, n_chunks):
    T = q.shape[0]
    nbh = T // block_t // n_chunks
    off = chunk * nbh
    cs_spec = pl.BlockSpec((block_t, HEAD_SIZE), lambda i: (i, 0))
    q_spec = pl.BlockSpec((block_t, q.shape[1]), lambda i: (i + off, 0))
    k_spec = pl.BlockSpec((block_t, k.shape[1]), lambda i: (i + off, 0))
    in_specs = [cs_spec, cs_spec, q_spec, k_spec]
    args = [cos_g, sin_g, q, k]
    aliases = {}
    if prev is not None:
        in_specs += [pl.BlockSpec(memory_space=pl.ANY),
                     pl.BlockSpec(memory_space=pl.ANY)]
        args += list(prev)
        aliases = {4: 0, 5: 1}
    return pl.pallas_call(
        _apply_body,
        grid=(nbh,),
        in_specs=in_specs,
        out_specs=[q_spec, k_spec],
        out_shape=[jax.ShapeDtypeStruct(q.shape, jnp.float32),
                   jax.ShapeDtypeStruct(k.shape, jnp.float32)],
        input_output_aliases=aliases,
        compiler_params=pltpu.CompilerParams(
            dimension_semantics=("arbitrary",)),
    )(*args)


@jax.jit
def kernel(positions, query, key, cos_cache, sin_cache):
    T = positions.shape[0]
    pos2d = positions.astype(jnp.int32).reshape(T // 128, 128)
    n_chunks = 2
    hr = pos2d.shape[0] // n_chunks
    gathered = [
        _sc_gather(pos2d[i * hr:(i + 1) * hr], cos_cache, sin_cache)
        for i in range(n_chunks)
    ]
    outs = None
    for i, (cg, sg) in enumerate(gathered):
        outs = _tc_apply_chunk(cg, sg, query, key, outs,
                               block_t=512, chunk=i, n_chunks=n_chunks)
    return outs


# speedup vs baseline: 1.0038x; 1.0038x over previous
import functools

import jax
import jax.numpy as jnp
from jax import lax
from jax.experimental import pallas as pl
from jax.experimental.pallas import tpu as pltpu
from jax.experimental.pallas import tpu_sc as plsc

HEAD_SIZE = 128
HALF = 64
NUM_Q_HEADS = 32
NUM_KV_HEADS = 8

_NC, _NS = 2, 16
_NW = _NC * _NS


def _sc_gather(positions, cos_cache, sin_cache, tokens, offset):
    rows = tokens // _NW
    assert rows <= 128 and rows % 8 == 0

    def body(pos_hbm, cos_hbm, sin_hbm, cos_out, sin_out,
             idx_v, cbuf, sbuf, sem):
        wid = lax.axis_index("s") * _NC + lax.axis_index("c")
        pltpu.sync_copy(pos_hbm.at[pl.ds(offset + wid * rows, rows)], idx_v)
        c1 = pltpu.async_copy(cos_hbm.at[idx_v], cbuf, sem)
        c2 = pltpu.async_copy(sin_hbm.at[idx_v], sbuf, sem)
        c1.wait()
        c2.wait()
        pltpu.sync_copy(cbuf, cos_out.at[pl.ds(wid * rows, rows)])
        pltpu.sync_copy(sbuf, sin_out.at[pl.ds(wid * rows, rows)])

    mesh = plsc.VectorSubcoreMesh(core_axis_name="c", subcore_axis_name="s",
                                  num_cores=_NC, num_subcores=_NS)
    f = pl.kernel(
        body,
        out_type=[jax.ShapeDtypeStruct((tokens, HEAD_SIZE), jnp.float32),
                  jax.ShapeDtypeStruct((tokens, HEAD_SIZE), jnp.float32)],
        mesh=mesh,
        scratch_types=[
            pltpu.VMEM((rows,), jnp.int32),
            pltpu.VMEM((rows, HEAD_SIZE), jnp.float32),
            pltpu.VMEM((rows, HEAD_SIZE), jnp.float32),
            pltpu.SemaphoreType.DMA,
        ],
    )
    return f(positions, cos_cache, sin_cache)


def _apply_body(cos_ref, sin_ref, q_ref, k_ref, *rest):
    qo_ref, ko_ref = rest[-2], rest[-1]
    c = cos_ref[...][:, :HALF]
    s = sin_ref[...][:, :HALF]
    cc = jnp.concatenate([c, c], axis=1)
    ss = jnp.concatenate([-s, s], axis=1)
    for x_ref, o_ref, heads in ((q_ref, qo_ref, NUM_Q_HEADS),
                                (k_ref, ko_ref, NUM_KV_HEADS)):
        for h in range(heads):
            x = x_ref[:, h * HEAD_SIZE:(h + 1) * HEAD_SIZE]
            xs = jnp.concatenate([x[:, HALF:], x[:, :HALF]], axis=1)
            o_ref[:, h * HEAD_SIZE:(h + 1) * HEAD_SIZE] = x * cc + xs * ss


def _tc_apply_chunk(cos_g, sin_g, q, k, prev, block_t, chunk, n_chunks):
    T = q.shape[0]
    nbh = T // block_t // n_chunks
    off = chunk * nbh
    cs_spec = pl.BlockSpec((block_t, HEAD_SIZE), lambda i: (i, 0))
    q_spec = pl.BlockSpec((block_t, q.shape[1]), lambda i: (i + off, 0))
    k_spec = pl.BlockSpec((block_t, k.shape[1]), lambda i: (i + off, 0))
    in_specs = [cs_spec, cs_spec, q_spec, k_spec]
    args = [cos_g, sin_g, q, k]
    aliases = {}
    if prev is not None:
        in_specs += [pl.BlockSpec(memory_space=pl.ANY),
                     pl.BlockSpec(memory_space=pl.ANY)]
        args += list(prev)
        aliases = {4: 0, 5: 1}
    return pl.pallas_call(
        _apply_body,
        grid=(nbh,),
        in_specs=in_specs,
        out_specs=[q_spec, k_spec],
        out_shape=[jax.ShapeDtypeStruct(q.shape, jnp.float32),
                   jax.ShapeDtypeStruct(k.shape, jnp.float32)],
        input_output_aliases=aliases,
        compiler_params=pltpu.CompilerParams(
            dimension_semantics=("arbitrary",)),
    )(*args)


@jax.jit
def kernel(positions, query, key, cos_cache, sin_cache):
    T = positions.shape[0]
    n_chunks = 2
    tc = T // n_chunks
    gathered = [
        _sc_gather(positions, cos_cache, sin_cache, tokens=tc, offset=i * tc)
        for i in range(n_chunks)
    ]
    outs = None
    for i, (cg, sg) in enumerate(gathered):
        outs = _tc_apply_chunk(cg, sg, query, key, outs,
                               block_t=512, chunk=i, n_chunks=n_chunks)
    return outs


# speedup vs baseline: 1.0050x; 1.0012x over previous
import functools

import jax
import jax.numpy as jnp
from jax import lax
from jax.experimental import pallas as pl
from jax.experimental.pallas import tpu as pltpu
from jax.experimental.pallas import tpu_sc as plsc

HEAD_SIZE = 128
HALF = 64
NUM_Q_HEADS = 32
NUM_KV_HEADS = 8

_NC, _NS = 2, 16
_NW = _NC * _NS


def _sc_gather(positions, cos_cache, sin_cache, tokens, offset):
    rows = tokens // _NW
    ns = -(-rows // 128)
    p = rows // ns
    assert p * ns == rows and p % 8 == 0 and p <= 128

    def body(pos_hbm, cos_hbm, sin_hbm, cos_out, sin_out,
             idx_v, cbuf, sbuf, sem):
        wid = lax.axis_index("s") * _NC + lax.axis_index("c")
        for j in range(ns):
            pltpu.sync_copy(
                pos_hbm.at[pl.ds(offset + wid * rows + j * p, p)],
                idx_v.at[j])
        copies = []
        for j in range(ns):
            copies.append(pltpu.async_copy(
                cos_hbm.at[idx_v.at[j]], cbuf.at[pl.ds(j * p, p)], sem))
            copies.append(pltpu.async_copy(
                sin_hbm.at[idx_v.at[j]], sbuf.at[pl.ds(j * p, p)], sem))
        for c in copies:
            c.wait()
        pltpu.sync_copy(cbuf, cos_out.at[pl.ds(wid * rows, rows)])
        pltpu.sync_copy(sbuf, sin_out.at[pl.ds(wid * rows, rows)])

    mesh = plsc.VectorSubcoreMesh(core_axis_name="c", subcore_axis_name="s",
                                  num_cores=_NC, num_subcores=_NS)
    f = pl.kernel(
        body,
        out_type=[jax.ShapeDtypeStruct((tokens, HEAD_SIZE), jnp.float32),
                  jax.ShapeDtypeStruct((tokens, HEAD_SIZE), jnp.float32)],
        mesh=mesh,
        scratch_types=[
            pltpu.VMEM((ns, p), jnp.int32),
            pltpu.VMEM((rows, HEAD_SIZE), jnp.float32),
            pltpu.VMEM((rows, HEAD_SIZE), jnp.float32),
            pltpu.SemaphoreType.DMA,
        ],
    )
    return f(positions, cos_cache, sin_cache)


def _apply_body(cos_ref, sin_ref, q_ref, k_ref, *rest):
    qo_ref, ko_ref = rest[-2], rest[-1]
    c = cos_ref[...][:, :HALF]
    s = sin_ref[...][:, :HALF]
    cc = jnp.concatenate([c, c], axis=1)
    ss = jnp.concatenate([-s, s], axis=1)
    for x_ref, o_ref, heads in ((q_ref, qo_ref, NUM_Q_HEADS),
                                (k_ref, ko_ref, NUM_KV_HEADS)):
        for h in range(heads):
            x = x_ref[:, h * HEAD_SIZE:(h + 1) * HEAD_SIZE]
            xs = jnp.concatenate([x[:, HALF:], x[:, :HALF]], axis=1)
            o_ref[:, h * HEAD_SIZE:(h + 1) * HEAD_SIZE] = x * cc + xs * ss


def _tc_apply_chunk(cos_g, sin_g, q, k, prev, block_t, tok_offset):
    nbh = cos_g.shape[0] // block_t
    off = tok_offset // block_t
    cs_spec = pl.BlockSpec((block_t, HEAD_SIZE), lambda i: (i, 0))
    q_spec = pl.BlockSpec((block_t, q.shape[1]), lambda i: (i + off, 0))
    k_spec = pl.BlockSpec((block_t, k.shape[1]), lambda i: (i + off, 0))
    in_specs = [cs_spec, cs_spec, q_spec, k_spec]
    args = [cos_g, sin_g, q, k]
    aliases = {}
    if prev is not None:
        in_specs += [pl.BlockSpec(memory_space=pl.ANY),
                     pl.BlockSpec(memory_space=pl.ANY)]
        args += list(prev)
        aliases = {4: 0, 5: 1}
    return pl.pallas_call(
        _apply_body,
        grid=(nbh,),
        in_specs=in_specs,
        out_specs=[q_spec, k_spec],
        out_shape=[jax.ShapeDtypeStruct(q.shape, jnp.float32),
                   jax.ShapeDtypeStruct(k.shape, jnp.float32)],
        input_output_aliases=aliases,
        compiler_params=pltpu.CompilerParams(
            dimension_semantics=("arbitrary",)),
    )(*args)


@jax.jit
def kernel(positions, query, key, cos_cache, sin_cache):
    T = positions.shape[0]
    chunks = (2048, 6144)
    offs = [sum(chunks[:i]) for i in range(len(chunks))]
    gathered = [
        _sc_gather(positions, cos_cache, sin_cache, tokens=c, offset=o)
        for c, o in zip(chunks, offs)
    ]
    outs = None
    for o, (cg, sg) in zip(offs, gathered):
        outs = _tc_apply_chunk(cg, sg, query, key, outs,
                               block_t=512, tok_offset=o)
    return outs


# speedup vs baseline: 1.0197x; 1.0146x over previous
import functools

import jax
import jax.numpy as jnp
from jax import lax
from jax.experimental import pallas as pl
from jax.experimental.pallas import tpu as pltpu
from jax.experimental.pallas import tpu_sc as plsc

HEAD_SIZE = 128
HALF = 64
NUM_Q_HEADS = 32
NUM_KV_HEADS = 8

_NC, _NS = 2, 16
_NW = _NC * _NS


def _sc_gather(positions, cos_cache, sin_cache, tokens, offset):
    rows = tokens // _NW
    ns = -(-rows // 128)
    p = rows // ns
    assert p * ns == rows and p % 8 == 0 and p <= 128

    def body(pos_hbm, cos_hbm, sin_hbm, cos_out, sin_out,
             idx_v, cbuf, sbuf, sem):
        wid = lax.axis_index("s") * _NC + lax.axis_index("c")
        for j in range(ns):
            pltpu.sync_copy(
                pos_hbm.at[pl.ds(offset + wid * rows + j * p, p)],
                idx_v.at[j])
        copies = []
        for j in range(ns):
            copies.append(pltpu.async_copy(
                cos_hbm.at[idx_v.at[j]], cbuf.at[pl.ds(j * p, p)], sem))
            copies.append(pltpu.async_copy(
                sin_hbm.at[idx_v.at[j]], sbuf.at[pl.ds(j * p, p)], sem))
        for c in copies:
            c.wait()
        pltpu.sync_copy(cbuf, cos_out.at[pl.ds(wid * rows, rows)])
        pltpu.sync_copy(sbuf, sin_out.at[pl.ds(wid * rows, rows)])

    mesh = plsc.VectorSubcoreMesh(core_axis_name="c", subcore_axis_name="s",
                                  num_cores=_NC, num_subcores=_NS)
    f = pl.kernel(
        body,
        out_type=[jax.ShapeDtypeStruct((tokens, HEAD_SIZE), jnp.float32),
                  jax.ShapeDtypeStruct((tokens, HEAD_SIZE), jnp.float32)],
        mesh=mesh,
        scratch_types=[
            pltpu.VMEM((ns, p), jnp.int32),
            pltpu.VMEM((rows, HEAD_SIZE), jnp.float32),
            pltpu.VMEM((rows, HEAD_SIZE), jnp.float32),
            pltpu.SemaphoreType.DMA,
        ],
    )
    return f(positions, cos_cache, sin_cache)


def _apply_body(cos_ref, sin_ref, q_ref, k_ref, *rest):
    qo_ref, ko_ref = rest[-2], rest[-1]
    c = cos_ref[...][:, :HALF]
    s = sin_ref[...][:, :HALF]
    cc = jnp.concatenate([c, c], axis=1)
    ss = jnp.concatenate([-s, s], axis=1)
    for x_ref, o_ref, heads in ((q_ref, qo_ref, NUM_Q_HEADS),
                                (k_ref, ko_ref, NUM_KV_HEADS)):
        for h in range(heads):
            x = x_ref[:, h * HEAD_SIZE:(h + 1) * HEAD_SIZE]
            xs = jnp.concatenate([x[:, HALF:], x[:, :HALF]], axis=1)
            o_ref[:, h * HEAD_SIZE:(h + 1) * HEAD_SIZE] = x * cc + xs * ss


def _tc_apply_chunk(cos_g, sin_g, q, k, prev, block_t, tok_offset):
    nbh = cos_g.shape[0] // block_t
    off = tok_offset // block_t
    cs_spec = pl.BlockSpec((block_t, HEAD_SIZE), lambda i: (i, 0))
    q_spec = pl.BlockSpec((block_t, q.shape[1]), lambda i: (i + off, 0))
    k_spec = pl.BlockSpec((block_t, k.shape[1]), lambda i: (i + off, 0))
    in_specs = [cs_spec, cs_spec, q_spec, k_spec]
    args = [cos_g, sin_g, q, k]
    aliases = {}
    if prev is not None:
        in_specs += [pl.BlockSpec(memory_space=pl.ANY),
                     pl.BlockSpec(memory_space=pl.ANY)]
        args += list(prev)
        aliases = {4: 0, 5: 1}
    return pl.pallas_call(
        _apply_body,
        grid=(nbh,),
        in_specs=in_specs,
        out_specs=[q_spec, k_spec],
        out_shape=[jax.ShapeDtypeStruct(q.shape, jnp.float32),
                   jax.ShapeDtypeStruct(k.shape, jnp.float32)],
        input_output_aliases=aliases,
        compiler_params=pltpu.CompilerParams(
            dimension_semantics=("arbitrary",)),
    )(*args)


@jax.jit
def kernel(positions, query, key, cos_cache, sin_cache):
    T = positions.shape[0]
    chunks = (8192,)
    offs = [sum(chunks[:i]) for i in range(len(chunks))]
    gathered = [
        _sc_gather(positions, cos_cache, sin_cache, tokens=c, offset=o)
        for c, o in zip(chunks, offs)
    ]
    outs = None
    for o, (cg, sg) in zip(offs, gathered):
        outs = _tc_apply_chunk(cg, sg, query, key, outs,
                               block_t=512, tok_offset=o)
    return outs


# speedup vs baseline: 1.0201x; 1.0004x over previous
import functools

import jax
import jax.numpy as jnp
from jax import lax
from jax.experimental import pallas as pl
from jax.experimental.pallas import tpu as pltpu
from jax.experimental.pallas import tpu_sc as plsc

HEAD_SIZE = 128
HALF = 64
NUM_Q_HEADS = 32
NUM_KV_HEADS = 8

_NC, _NS = 2, 16
_NW = _NC * _NS


def _sc_gather(positions, cos_cache, sin_cache, tokens, offset):
    rows = tokens // _NW
    ns = -(-rows // 128)
    p = rows // ns
    assert p * ns == rows and p % 8 == 0 and p <= 128

    def body(pos_hbm, cos_hbm, sin_hbm, cos_out, sin_out,
             idx_v, cbuf, sbuf, sem):
        wid = lax.axis_index("s") * _NC + lax.axis_index("c")
        for j in range(ns):
            pltpu.sync_copy(
                pos_hbm.at[pl.ds(offset + wid * rows + j * p, p)],
                idx_v.at[j])
        copies = []
        for j in range(ns):
            copies.append(pltpu.async_copy(
                cos_hbm.at[idx_v.at[j]], cbuf.at[pl.ds(j * p, p)], sem))
            copies.append(pltpu.async_copy(
                sin_hbm.at[idx_v.at[j]], sbuf.at[pl.ds(j * p, p)], sem))
        for c in copies:
            c.wait()
        pltpu.sync_copy(cbuf, cos_out.at[pl.ds(wid * rows, rows)])
        pltpu.sync_copy(sbuf, sin_out.at[pl.ds(wid * rows, rows)])

    mesh = plsc.VectorSubcoreMesh(core_axis_name="c", subcore_axis_name="s",
                                  num_cores=_NC, num_subcores=_NS)
    f = pl.kernel(
        body,
        out_type=[jax.ShapeDtypeStruct((tokens, HEAD_SIZE), jnp.float32),
                  jax.ShapeDtypeStruct((tokens, HEAD_SIZE), jnp.float32)],
        mesh=mesh,
        scratch_types=[
            pltpu.VMEM((ns, p), jnp.int32),
            pltpu.VMEM((rows, HEAD_SIZE), jnp.float32),
            pltpu.VMEM((rows, HEAD_SIZE), jnp.float32),
            pltpu.SemaphoreType.DMA,
        ],
    )
    return f(positions, cos_cache, sin_cache)


def _apply_body(cos_ref, sin_ref, q_ref, k_ref, *rest):
    qo_ref, ko_ref = rest[-2], rest[-1]
    c = cos_ref[...][:, :HALF]
    s = sin_ref[...][:, :HALF]
    cc = jnp.concatenate([c, c], axis=1)
    ss = jnp.concatenate([-s, s], axis=1)
    for x_ref, o_ref, heads in ((q_ref, qo_ref, NUM_Q_HEADS),
                                (k_ref, ko_ref, NUM_KV_HEADS)):
        for h in range(heads):
            x = x_ref[:, h * HEAD_SIZE:(h + 1) * HEAD_SIZE]
            xs = jnp.concatenate([x[:, HALF:], x[:, :HALF]], axis=1)
            o_ref[:, h * HEAD_SIZE:(h + 1) * HEAD_SIZE] = x * cc + xs * ss


def _tc_apply_chunk(cos_g, sin_g, q, k, prev, block_t, tok_offset):
    nbh = cos_g.shape[0] // block_t
    off = tok_offset // block_t
    cs_spec = pl.BlockSpec((block_t, HEAD_SIZE), lambda i: (i, 0))
    q_spec = pl.BlockSpec((block_t, q.shape[1]), lambda i: (i + off, 0))
    k_spec = pl.BlockSpec((block_t, k.shape[1]), lambda i: (i + off, 0))
    in_specs = [cs_spec, cs_spec, q_spec, k_spec]
    args = [cos_g, sin_g, q, k]
    aliases = {}
    if prev is not None:
        in_specs += [pl.BlockSpec(memory_space=pl.ANY),
                     pl.BlockSpec(memory_space=pl.ANY)]
        args += list(prev)
        aliases = {4: 0, 5: 1}
    return pl.pallas_call(
        _apply_body,
        grid=(nbh,),
        in_specs=in_specs,
        out_specs=[q_spec, k_spec],
        out_shape=[jax.ShapeDtypeStruct(q.shape, jnp.float32),
                   jax.ShapeDtypeStruct(k.shape, jnp.float32)],
        input_output_aliases=aliases,
        compiler_params=pltpu.CompilerParams(
            dimension_semantics=("arbitrary",),
            vmem_limit_bytes=110 * 1024 * 1024),
    )(*args)


@jax.jit
def kernel(positions, query, key, cos_cache, sin_cache):
    T = positions.shape[0]
    chunks = (8192,)
    offs = [sum(chunks[:i]) for i in range(len(chunks))]
    gathered = [
        _sc_gather(positions, cos_cache, sin_cache, tokens=c, offset=o)
        for c, o in zip(chunks, offs)
    ]
    outs = None
    for o, (cg, sg) in zip(offs, gathered):
        outs = _tc_apply_chunk(cg, sg, query, key, outs,
                               block_t=512, tok_offset=o)
    return outs


# speedup vs baseline: 1.0202x; 1.0001x over previous
import jax
import jax.numpy as jnp
from jax import lax
from jax.experimental import pallas as pl
from jax.experimental.pallas import tpu as pltpu
from jax.experimental.pallas import tpu_sc as plsc

HEAD_SIZE = 128
HALF = 64
NUM_Q_HEADS = 32
NUM_KV_HEADS = 8

_NC, _NS = 2, 16
_NW = _NC * _NS


def _sc_gather(positions, comb_table, tokens, offset):
    rows = tokens // _NW
    ns = -(-rows // 128)
    p = rows // ns
    assert p * ns == rows and p % 8 == 0 and p <= 128

    def body(pos_hbm, tab_hbm, out_hbm, idx_v, buf, sem):
        wid = lax.axis_index("s") * _NC + lax.axis_index("c")
        for j in range(ns):
            pltpu.sync_copy(
                pos_hbm.at[pl.ds(offset + wid * rows + j * p, p)],
                idx_v.at[j])
        copies = [
            pltpu.async_copy(
                tab_hbm.at[idx_v.at[j]], buf.at[pl.ds(j * p, p)], sem)
            for j in range(ns)
        ]
        for c in copies:
            c.wait()
        pltpu.sync_copy(buf, out_hbm.at[pl.ds(wid * rows, rows)])

    mesh = plsc.VectorSubcoreMesh(core_axis_name="c", subcore_axis_name="s",
                                  num_cores=_NC, num_subcores=_NS)
    f = pl.kernel(
        body,
        out_type=jax.ShapeDtypeStruct((tokens, HEAD_SIZE), jnp.float32),
        mesh=mesh,
        scratch_types=[
            pltpu.VMEM((ns, p), jnp.int32),
            pltpu.VMEM((rows, HEAD_SIZE), jnp.float32),
            pltpu.SemaphoreType.DMA,
        ],
    )
    return f(positions, comb_table)


def _apply_body(cs_ref, q_ref, k_ref, *rest):
    qo_ref, ko_ref = rest[-2], rest[-1]
    cs = cs_ref[...]
    c = cs[:, :HALF]
    s = cs[:, HALF:]
    cc = jnp.concatenate([c, c], axis=1)
    ss = jnp.concatenate([-s, s], axis=1)
    for x_ref, o_ref, heads in ((q_ref, qo_ref, NUM_Q_HEADS),
                                (k_ref, ko_ref, NUM_KV_HEADS)):
        for h in range(heads):
            x = x_ref[:, h * HEAD_SIZE:(h + 1) * HEAD_SIZE]
            xs = jnp.concatenate([x[:, HALF:], x[:, :HALF]], axis=1)
            o_ref[:, h * HEAD_SIZE:(h + 1) * HEAD_SIZE] = x * cc + xs * ss


def _tc_apply_chunk(cs_g, q, k, prev, block_t, tok_offset):
    nbh = cs_g.shape[0] // block_t
    off = tok_offset // block_t
    cs_spec = pl.BlockSpec((block_t, HEAD_SIZE), lambda i: (i, 0))
    q_spec = pl.BlockSpec((block_t, q.shape[1]), lambda i: (i + off, 0))
    k_spec = pl.BlockSpec((block_t, k.shape[1]), lambda i: (i + off, 0))
    in_specs = [cs_spec, q_spec, k_spec]
    args = [cs_g, q, k]
    aliases = {}
    if prev is not None:
        in_specs += [pl.BlockSpec(memory_space=pl.ANY),
                     pl.BlockSpec(memory_space=pl.ANY)]
        args += list(prev)
        aliases = {3: 0, 4: 1}
    return pl.pallas_call(
        _apply_body,
        grid=(nbh,),
        in_specs=in_specs,
        out_specs=[q_spec, k_spec],
        out_shape=[jax.ShapeDtypeStruct(q.shape, jnp.float32),
                   jax.ShapeDtypeStruct(k.shape, jnp.float32)],
        input_output_aliases=aliases,
        compiler_params=pltpu.CompilerParams(
            dimension_semantics=("arbitrary",)),
    )(*args)


@jax.jit
def kernel(positions, query, key, cos_cache, sin_cache):
    T = positions.shape[0]
    comb = jnp.concatenate([cos_cache[:, :HALF], sin_cache[:, :HALF]], axis=1)
    chunks = (T,)
    offs = [sum(chunks[:i]) for i in range(len(chunks))]
    gathered = [
        _sc_gather(positions, comb, tokens=c, offset=o)
        for c, o in zip(chunks, offs)
    ]
    outs = None
    for o, cs_g in zip(offs, gathered):
        outs = _tc_apply_chunk(cs_g, query, key, outs,
                               block_t=512, tok_offset=o)
    return outs
